# Initial kernel scaffold; baseline (speedup 1.0000x reference)
import jax
import jax.numpy as jnp
from jax.experimental import pallas as pl


def kernel(x, table, gamma, beta):
    raise NotImplementedError("write your pallas kernel here")


# speedup vs baseline: 1.8969x; 1.8969x over previous
import functools

import jax
import jax.numpy as jnp
from jax import lax
from jax.experimental import pallas as pl
from jax.experimental.pallas import tpu as pltpu
from jax.experimental.pallas import tpu_sc as plsc

NC = 2
NS = 16
NW = NC * NS
D = 128
L16 = 16
CHUNK = 128


def _lane_allsum(v):
    dnums = lax.GatherDimensionNumbers(
        offset_dims=(), collapsed_slice_dims=(0,), start_index_map=(0,))
    lanes = lax.iota(jnp.int32, L16)
    for s in (1, 2, 4, 8):
        idx = jnp.bitwise_xor(lanes, jnp.int32(s)).reshape(L16, 1)
        v = v + lax.gather(v, idx, dnums, (1,),
                           mode=lax.GatherScatterMode.PROMISE_IN_BOUNDS)
    return v


def _rsqrt_vec(v):
    i = lax.bitcast_convert_type(v, jnp.int32)
    i = jnp.int32(0x5F3759DF) - lax.shift_right_arithmetic(i, 1)
    y = lax.bitcast_convert_type(i, jnp.float32)
    for _ in range(3):
        y = y * (1.5 - 0.5 * v * y * y)
    return y


@functools.lru_cache(maxsize=None)
def _make_sc_kernel(B):
    rows_per_w = B // NW
    nchunk = rows_per_w // CHUNK
    mesh = plsc.VectorSubcoreMesh(core_axis_name="c", subcore_axis_name="s")

    @functools.partial(
        pl.kernel,
        mesh=mesh,
        out_type=jax.ShapeDtypeStruct((B, D), jnp.float32),
        scratch_types=[
            pltpu.VMEM((CHUNK,), jnp.int32),
            pltpu.VMEM((CHUNK, D), jnp.float32),
            pltpu.VMEM((D,), jnp.float32),
            pltpu.VMEM((D,), jnp.float32),
            pltpu.SemaphoreType.DMA,
        ],
    )
    def body(x_hbm, tab_hbm, gamma_hbm, beta_hbm, out_hbm,
             idx_v, rows_v, g_v, b_v, sem):
        wid = lax.axis_index("s") * NC + lax.axis_index("c")
        base = wid * rows_per_w
        pltpu.sync_copy(gamma_hbm, g_v)
        pltpu.sync_copy(beta_hbm, b_v)
        gs = tuple(g_v[pl.ds(L16 * k, L16)] for k in range(D // L16))
        bs = tuple(b_v[pl.ds(L16 * k, L16)] for k in range(D // L16))

        def do_chunk(ci, carry):
            rbase = base + ci * CHUNK
            pltpu.sync_copy(x_hbm.at[pl.ds(rbase, CHUNK)], idx_v)
            pltpu.async_copy(tab_hbm.at[idx_v], rows_v, sem).wait()

            def do_row(r, carry2):
                cgs, cbs = carry2
                v = [rows_v[r, pl.ds(L16 * k, L16)] for k in range(D // L16)]
                sv = (((v[0] + v[1]) + (v[2] + v[3]))
                      + ((v[4] + v[5]) + (v[6] + v[7])))
                q = [vv * vv for vv in v]
                qv = (((q[0] + q[1]) + (q[2] + q[3]))
                      + ((q[4] + q[5]) + (q[6] + q[7])))
                mb = _lane_allsum(sv) * (1.0 / D)
                var = _lane_allsum(qv) * (1.0 / D) - mb * mb
                rstd = _rsqrt_vec(var + 1e-5)
                for k in range(D // L16):
                    rows_v[r, pl.ds(L16 * k, L16)] = (
                        (v[k] - mb) * rstd * cgs[k] + cbs[k])
                return carry2

            lax.fori_loop(0, CHUNK, do_row, carry)
            pltpu.sync_copy(rows_v, out_hbm.at[pl.ds(rbase, CHUNK)])
            return carry

        lax.fori_loop(0, nchunk, do_chunk, (gs, bs))

    return body


def kernel(x, table, gamma, beta):
    bsz, seq = x.shape
    xi = x.reshape(-1).astype(jnp.int32)
    out = _make_sc_kernel(xi.shape[0])(xi, table, gamma, beta)
    return out.reshape(bsz, seq, D)


# speedup vs baseline: 3.2643x; 1.7208x over previous
import functools

import jax
import jax.numpy as jnp
from jax import lax
from jax.experimental import pallas as pl
from jax.experimental.pallas import tpu as pltpu
from jax.experimental.pallas import tpu_sc as plsc

NC = 2
NS = 16
NW = NC * NS
D = 128
L16 = 16
NK = D // L16
CHUNK = 64
NBUF = 4


def _lane_allsum(v):
    dnums = lax.GatherDimensionNumbers(
        offset_dims=(), collapsed_slice_dims=(0,), start_index_map=(0,))
    lanes = lax.iota(jnp.int32, L16)
    for s in (1, 2, 4, 8):
        idx = jnp.bitwise_xor(lanes, jnp.int32(s)).reshape(L16, 1)
        v = v + lax.gather(v, idx, dnums, (1,),
                           mode=lax.GatherScatterMode.PROMISE_IN_BOUNDS)
    return v


def _rsqrt_vec(v):
    i = lax.bitcast_convert_type(v, jnp.int32)
    i = jnp.int32(0x5F3759DF) - lax.shift_right_arithmetic(i, 1)
    y = lax.bitcast_convert_type(i, jnp.float32)
    vh = 0.5 * v
    for _ in range(2):
        y = y * (1.5 - vh * y * y)
    return y


def _tree8(v):
    return (((v[0] + v[1]) + (v[2] + v[3])) + ((v[4] + v[5]) + (v[6] + v[7])))


def _ln_rows(rows_ref, r, cgs, cbs):
    v = [rows_ref[r, pl.ds(L16 * k, L16)] for k in range(NK)]
    mb = _lane_allsum(_tree8(v)) * (1.0 / D)
    var = _lane_allsum(_tree8([x * x for x in v])) * (1.0 / D) - mb * mb
    rstd = _rsqrt_vec(var + 1e-5)
    for k in range(NK):
        rows_ref[r, pl.ds(L16 * k, L16)] = (v[k] - mb) * rstd * cgs[k] + cbs[k]


@functools.lru_cache(maxsize=None)
def _make_sc_kernel(B):
    rows_per_w = B // NW
    nchunk = rows_per_w // CHUNK
    nsuper = nchunk // NBUF
    mesh = plsc.VectorSubcoreMesh(core_axis_name="c", subcore_axis_name="s")

    @functools.partial(
        pl.kernel,
        mesh=mesh,
        out_type=jax.ShapeDtypeStruct((B, D), jnp.float32),
        scratch_types=[
            pltpu.VMEM((rows_per_w,), jnp.int32),
            pltpu.VMEM((CHUNK, D), jnp.float32),
            pltpu.VMEM((CHUNK, D), jnp.float32),
            pltpu.VMEM((CHUNK, D), jnp.float32),
            pltpu.VMEM((CHUNK, D), jnp.float32),
            pltpu.VMEM((D,), jnp.float32),
            pltpu.VMEM((D,), jnp.float32),
            pltpu.SemaphoreType.DMA,
            pltpu.SemaphoreType.DMA,
            pltpu.SemaphoreType.DMA,
            pltpu.SemaphoreType.DMA,
            pltpu.SemaphoreType.DMA,
            pltpu.SemaphoreType.DMA,
            pltpu.SemaphoreType.DMA,
            pltpu.SemaphoreType.DMA,
        ],
    )
    def body(x_hbm, tab_hbm, gamma_hbm, beta_hbm, out_hbm,
             idx_all, r0, r1, r2, r3, g_v, b_v,
             g0, g1, g2, g3, o0, o1, o2, o3):
        wid = lax.axis_index("s") * NC + lax.axis_index("c")
        base = wid * rows_per_w
        bufs = (r0, r1, r2, r3)
        semg = (g0, g1, g2, g3)
        semo = (o0, o1, o2, o3)

        pltpu.sync_copy(gamma_hbm, g_v)
        pltpu.sync_copy(beta_hbm, b_v)
        pltpu.sync_copy(x_hbm.at[pl.ds(base, rows_per_w)], idx_all)
        gs = tuple(g_v[pl.ds(L16 * k, L16)] for k in range(NK))
        bs = tuple(b_v[pl.ds(L16 * k, L16)] for k in range(NK))

        def start_gather(c, buf, sem):
            pltpu.async_copy(
                tab_hbm.at[idx_all.at[pl.ds(c * CHUNK, CHUNK)]], buf, sem)

        start_gather(jnp.int32(0), bufs[0], semg[0])

        def super_body(ci, carry):
            for j in range(NBUF):
                c = ci * NBUF + j
                jn = (j + 1) % NBUF
                pltpu.make_async_copy(
                    tab_hbm.at[pl.ds(0, CHUNK)], bufs[j], semg[j]).wait()
                out_wait = lambda: pltpu.make_async_copy(
                    bufs[jn], out_hbm.at[pl.ds(0, CHUNK)], semo[jn]).wait()
                if j < NBUF - 1:
                    pl.when(ci > 0)(out_wait)
                else:
                    out_wait()
                cn = jnp.minimum(c + 1, nchunk - 1)
                start_gather(cn, bufs[jn], semg[jn])

                def ln_pair(p, c2, _buf=bufs[j]):
                    cgs, cbs = c2
                    _ln_rows(_buf, 2 * p, cgs, cbs)
                    _ln_rows(_buf, 2 * p + 1, cgs, cbs)
                    return c2

                lax.fori_loop(0, CHUNK // 2, ln_pair, carry)
                pltpu.async_copy(
                    bufs[j], out_hbm.at[pl.ds(base + c * CHUNK, CHUNK)],
                    semo[j])
            return carry

        lax.fori_loop(0, nsuper, super_body, (gs, bs))

        for j in (1, 2, 3):
            pltpu.make_async_copy(
                bufs[j], out_hbm.at[pl.ds(0, CHUNK)], semo[j]).wait()
        pltpu.make_async_copy(
            tab_hbm.at[pl.ds(0, CHUNK)], bufs[0], semg[0]).wait()

    return body


def kernel(x, table, gamma, beta):
    bsz, seq = x.shape
    xi = x.reshape(-1).astype(jnp.int32)
    out = _make_sc_kernel(xi.shape[0])(xi, table, gamma, beta)
    return out.reshape(bsz, seq, D)


# speedup vs baseline: 7.2775x; 2.2295x over previous
import functools

import jax
import jax.numpy as jnp
from jax import lax
from jax.experimental import pallas as pl
from jax.experimental.pallas import tpu as pltpu
from jax.experimental.pallas import tpu_sc as plsc

NC = 2
NS = 16
NW = NC * NS
D = 128
L16 = 16
NK = D // L16
CHUNK = 64
NBUF = 4


def _lane_allsum(v):
    dnums = lax.GatherDimensionNumbers(
        offset_dims=(), collapsed_slice_dims=(0,), start_index_map=(0,))
    lanes = lax.iota(jnp.int32, L16)
    for s in (1, 2, 4, 8):
        idx = jnp.bitwise_xor(lanes, jnp.int32(s)).reshape(L16, 1)
        v = v + lax.gather(v, idx, dnums, (1,),
                           mode=lax.GatherScatterMode.PROMISE_IN_BOUNDS)
    return v


def _rsqrt_vec(v):
    i = lax.bitcast_convert_type(v, jnp.int32)
    i = jnp.int32(0x5F3759DF) - lax.shift_right_arithmetic(i, 1)
    y = lax.bitcast_convert_type(i, jnp.float32)
    vh = 0.5 * v
    for _ in range(2):
        y = y * (1.5 - vh * y * y)
    return y


def _tree8(v):
    return (((v[0] + v[1]) + (v[2] + v[3])) + ((v[4] + v[5]) + (v[6] + v[7])))


def _ln_rows(rows_ref, r, cgs, cbs):
    v = [rows_ref[r, pl.ds(L16 * k, L16)] for k in range(NK)]
    mb = _lane_allsum(_tree8(v)) * (1.0 / D)
    var = _lane_allsum(_tree8([x * x for x in v])) * (1.0 / D) - mb * mb
    rstd = _rsqrt_vec(var + 1e-5)
    for k in range(NK):
        rows_ref[r, pl.ds(L16 * k, L16)] = (v[k] - mb) * rstd * cgs[k] + cbs[k]


@functools.lru_cache(maxsize=None)
def _make_sc_kernel(B):
    rows_per_w = B // NW
    nchunk = rows_per_w // CHUNK
    nsuper = nchunk // NBUF
    mesh = plsc.VectorSubcoreMesh(core_axis_name="c", subcore_axis_name="s")

    @functools.partial(
        pl.kernel,
        mesh=mesh,
        out_type=jax.ShapeDtypeStruct((B, D), jnp.float32),
        scratch_types=[
            pltpu.VMEM((rows_per_w,), jnp.int32),
            pltpu.VMEM((CHUNK, D), jnp.float32),
            pltpu.VMEM((CHUNK, D), jnp.float32),
            pltpu.VMEM((CHUNK, D), jnp.float32),
            pltpu.VMEM((CHUNK, D), jnp.float32),
            pltpu.VMEM((D,), jnp.float32),
            pltpu.VMEM((D,), jnp.float32),
            pltpu.SemaphoreType.DMA,
            pltpu.SemaphoreType.DMA,
            pltpu.SemaphoreType.DMA,
            pltpu.SemaphoreType.DMA,
            pltpu.SemaphoreType.DMA,
            pltpu.SemaphoreType.DMA,
            pltpu.SemaphoreType.DMA,
            pltpu.SemaphoreType.DMA,
        ],
    )
    def body(x_hbm, tab_hbm, gamma_hbm, beta_hbm, out_hbm,
             idx_all, r0, r1, r2, r3, g_v, b_v,
             g0, g1, g2, g3, o0, o1, o2, o3):
        wid = lax.axis_index("s") * NC + lax.axis_index("c")
        base = wid * rows_per_w
        bufs = (r0, r1, r2, r3)
        semg = (g0, g1, g2, g3)
        semo = (o0, o1, o2, o3)

        pltpu.sync_copy(gamma_hbm, g_v)
        pltpu.sync_copy(beta_hbm, b_v)
        pltpu.sync_copy(x_hbm.at[pl.ds(base, rows_per_w)], idx_all)
        gs = tuple(g_v[pl.ds(L16 * k, L16)] for k in range(NK))
        bs = tuple(b_v[pl.ds(L16 * k, L16)] for k in range(NK))

        def start_gather(c, buf, sem):
            pltpu.async_copy(
                tab_hbm.at[idx_all.at[pl.ds(c * CHUNK, CHUNK)]], buf, sem)

        start_gather(jnp.int32(0), bufs[0], semg[0])

        def super_body(ci, carry):
            for j in range(NBUF):
                c = ci * NBUF + j
                jn = (j + 1) % NBUF
                pltpu.make_async_copy(
                    tab_hbm.at[pl.ds(0, CHUNK)], bufs[j], semg[j]).wait()
                out_wait = lambda: pltpu.make_async_copy(
                    bufs[jn], out_hbm.at[pl.ds(0, CHUNK)], semo[jn]).wait()
                if j < NBUF - 1:
                    pl.when(ci > 0)(out_wait)
                else:
                    out_wait()
                cn = jnp.minimum(c + 1, nchunk - 1)
                start_gather(cn, bufs[jn], semg[jn])

                def ln_pair(p, c2, _buf=bufs[j]):
                    cgs, cbs = c2
                    _ln_rows(_buf, 2 * p, cgs, cbs)
                    _ln_rows(_buf, 2 * p + 1, cgs, cbs)
                    return c2

                lax.fori_loop(0, CHUNK // 2, ln_pair, carry)
                pltpu.async_copy(
                    bufs[j], out_hbm.at[pl.ds(base + c * CHUNK, CHUNK)],
                    semo[j])
            return carry

        lax.fori_loop(0, nsuper, super_body, (gs, bs))

        for j in (1, 2, 3):
            pltpu.make_async_copy(
                bufs[j], out_hbm.at[pl.ds(0, CHUNK)], semo[j]).wait()
        pltpu.make_async_copy(
            tab_hbm.at[pl.ds(0, CHUNK)], bufs[0], semg[0]).wait()

    return body


def kernel(x, table, gamma, beta):
    bsz, seq = x.shape
    xi = x.T.reshape(-1).astype(jnp.int32)
    out = _make_sc_kernel(xi.shape[0])(xi, table, gamma, beta)
    return out.reshape(seq, bsz, D).transpose(1, 0, 2)


# speedup vs baseline: 7.6865x; 1.0562x over previous
import functools

import jax
import jax.numpy as jnp
from jax import lax
from jax.experimental import pallas as pl
from jax.experimental.pallas import tpu as pltpu
from jax.experimental.pallas import tpu_sc as plsc

NC = 2
NS = 16
NW = NC * NS
D = 128
L16 = 16
NK = D // L16
CHUNK = 80
NBUF = 4


def _lane_allsum(v):
    dnums = lax.GatherDimensionNumbers(
        offset_dims=(), collapsed_slice_dims=(0,), start_index_map=(0,))
    lanes = lax.iota(jnp.int32, L16)
    for s in (1, 2, 4, 8):
        idx = jnp.bitwise_xor(lanes, jnp.int32(s)).reshape(L16, 1)
        v = v + lax.gather(v, idx, dnums, (1,),
                           mode=lax.GatherScatterMode.PROMISE_IN_BOUNDS)
    return v


def _rsqrt_vec(v):
    i = lax.bitcast_convert_type(v, jnp.int32)
    i = jnp.int32(0x5F3759DF) - lax.shift_right_arithmetic(i, 1)
    y = lax.bitcast_convert_type(i, jnp.float32)
    vh = 0.5 * v
    return y * (1.5 - vh * y * y)


def _tree8(v):
    return (((v[0] + v[1]) + (v[2] + v[3])) + ((v[4] + v[5]) + (v[6] + v[7])))


def _ln_rows(rows_ref, r, cgs, cbs):
    v = [rows_ref[r, pl.ds(L16 * k, L16)] for k in range(NK)]
    mb = _lane_allsum(_tree8(v)) * (1.0 / D)
    var = _lane_allsum(_tree8([x * x for x in v])) * (1.0 / D) - mb * mb
    rstd = _rsqrt_vec(var + 1e-5)
    for k in range(NK):
        rows_ref[r, pl.ds(L16 * k, L16)] = (v[k] - mb) * rstd * cgs[k] + cbs[k]


@functools.lru_cache(maxsize=None)
def _make_sc_kernel(B):
    rows_per_w = B // NW
    nchunk = rows_per_w // CHUNK
    nsuper = nchunk // NBUF
    mesh = plsc.VectorSubcoreMesh(core_axis_name="c", subcore_axis_name="s")

    @functools.partial(
        pl.kernel,
        mesh=mesh,
        out_type=jax.ShapeDtypeStruct((B, D), jnp.float32),
        scratch_types=[
            pltpu.VMEM((rows_per_w,), jnp.int32),
            pltpu.VMEM((CHUNK, D), jnp.float32),
            pltpu.VMEM((CHUNK, D), jnp.float32),
            pltpu.VMEM((CHUNK, D), jnp.float32),
            pltpu.VMEM((CHUNK, D), jnp.float32),
            pltpu.VMEM((D,), jnp.float32),
            pltpu.VMEM((D,), jnp.float32),
            pltpu.SemaphoreType.DMA,
            pltpu.SemaphoreType.DMA,
            pltpu.SemaphoreType.DMA,
            pltpu.SemaphoreType.DMA,
            pltpu.SemaphoreType.DMA,
            pltpu.SemaphoreType.DMA,
            pltpu.SemaphoreType.DMA,
            pltpu.SemaphoreType.DMA,
        ],
    )
    def body(x_hbm, tab_hbm, gamma_hbm, beta_hbm, out_hbm,
             idx_all, r0, r1, r2, r3, g_v, b_v,
             g0, g1, g2, g3, o0, o1, o2, o3):
        wid = lax.axis_index("s") * NC + lax.axis_index("c")
        base = wid * rows_per_w
        bufs = (r0, r1, r2, r3)
        semg = (g0, g1, g2, g3)
        semo = (o0, o1, o2, o3)

        pltpu.sync_copy(gamma_hbm, g_v)
        pltpu.sync_copy(beta_hbm, b_v)
        pltpu.sync_copy(x_hbm.at[pl.ds(base, rows_per_w)], idx_all)
        gs = tuple(g_v[pl.ds(L16 * k, L16)] for k in range(NK))
        bs = tuple(b_v[pl.ds(L16 * k, L16)] for k in range(NK))

        def start_gather(c, buf, sem):
            pltpu.async_copy(
                tab_hbm.at[idx_all.at[pl.ds(c * CHUNK, CHUNK)]], buf, sem)

        start_gather(jnp.int32(0), bufs[0], semg[0])

        def super_body(ci, carry):
            for j in range(NBUF):
                c = ci * NBUF + j
                jn = (j + 1) % NBUF
                pltpu.make_async_copy(
                    tab_hbm.at[pl.ds(0, CHUNK)], bufs[j], semg[j]).wait()
                out_wait = lambda: pltpu.make_async_copy(
                    bufs[jn], out_hbm.at[pl.ds(0, CHUNK)], semo[jn]).wait()
                if j < NBUF - 1:
                    pl.when(ci > 0)(out_wait)
                else:
                    out_wait()
                cn = jnp.minimum(c + 1, nchunk - 1)
                start_gather(cn, bufs[jn], semg[jn])

                def ln_pair(p, c2, _buf=bufs[j]):
                    cgs, cbs = c2
                    _ln_rows(_buf, 2 * p, cgs, cbs)
                    _ln_rows(_buf, 2 * p + 1, cgs, cbs)
                    return c2

                lax.fori_loop(0, CHUNK // 2, ln_pair, carry)
                pltpu.async_copy(
                    bufs[j], out_hbm.at[pl.ds(base + c * CHUNK, CHUNK)],
                    semo[j])
            return carry

        lax.fori_loop(0, nsuper, super_body, (gs, bs))

        for j in (1, 2, 3):
            pltpu.make_async_copy(
                bufs[j], out_hbm.at[pl.ds(0, CHUNK)], semo[j]).wait()
        pltpu.make_async_copy(
            tab_hbm.at[pl.ds(0, CHUNK)], bufs[0], semg[0]).wait()

    return body


def kernel(x, table, gamma, beta):
    bsz, seq = x.shape
    xi = x.T.reshape(-1).astype(jnp.int32)
    out = _make_sc_kernel(xi.shape[0])(xi, table, gamma, beta)
    return out.reshape(seq, bsz, D).transpose(1, 0, 2)


# speedup vs baseline: 8.0560x; 1.0481x over previous
import functools

import jax
import jax.numpy as jnp
from jax import lax
from jax.experimental import pallas as pl
from jax.experimental.pallas import tpu as pltpu
from jax.experimental.pallas import tpu_sc as plsc

NC = 2
NS = 16
NW = NC * NS
D = 128
L16 = 16
NK = D // L16
CHUNK = 80
NBUF = 4


def _lane_allsum(v):
    dnums = lax.GatherDimensionNumbers(
        offset_dims=(), collapsed_slice_dims=(0,), start_index_map=(0,))
    lanes = lax.iota(jnp.int32, L16)
    for s in (1, 2, 4, 8):
        idx = jnp.bitwise_xor(lanes, jnp.int32(s)).reshape(L16, 1)
        v = v + lax.gather(v, idx, dnums, (1,),
                           mode=lax.GatherScatterMode.PROMISE_IN_BOUNDS)
    return v


def _rsqrt_vec(v):
    i = lax.bitcast_convert_type(v, jnp.int32)
    i = jnp.int32(0x5F3759DF) - lax.shift_right_arithmetic(i, 1)
    y = lax.bitcast_convert_type(i, jnp.float32)
    vh = 0.5 * v
    return y * (1.5 - vh * y * y)


def _tree8(v):
    return (((v[0] + v[1]) + (v[2] + v[3])) + ((v[4] + v[5]) + (v[6] + v[7])))


def _ln_rows(rows_ref, r, cgs, cbs):
    v = [rows_ref[r, pl.ds(L16 * k, L16)] for k in range(NK)]
    mb = _lane_allsum(_tree8(v)) * (1.0 / D)
    var = _lane_allsum(_tree8([x * x for x in v])) * (1.0 / D) - mb * mb
    rstd = _rsqrt_vec(var + 1e-5)
    for k in range(NK):
        rows_ref[r, pl.ds(L16 * k, L16)] = (v[k] - mb) * rstd * cgs[k] + cbs[k]


@functools.lru_cache(maxsize=None)
def _make_sc_kernel(B):
    rows_per_w = B // NW
    nchunk = rows_per_w // CHUNK
    nsuper = nchunk // NBUF
    mesh = plsc.VectorSubcoreMesh(core_axis_name="c", subcore_axis_name="s")

    @functools.partial(
        pl.kernel,
        mesh=mesh,
        out_type=jax.ShapeDtypeStruct((B, D), jnp.float32),
        scratch_types=[
            pltpu.VMEM((rows_per_w,), jnp.int32),
            pltpu.VMEM((CHUNK, D), jnp.float32),
            pltpu.VMEM((CHUNK, D), jnp.float32),
            pltpu.VMEM((CHUNK, D), jnp.float32),
            pltpu.VMEM((CHUNK, D), jnp.float32),
            pltpu.VMEM((D,), jnp.float32),
            pltpu.VMEM((D,), jnp.float32),
            pltpu.SemaphoreType.DMA,
            pltpu.SemaphoreType.DMA,
            pltpu.SemaphoreType.DMA,
            pltpu.SemaphoreType.DMA,
            pltpu.SemaphoreType.DMA,
            pltpu.SemaphoreType.DMA,
            pltpu.SemaphoreType.DMA,
            pltpu.SemaphoreType.DMA,
        ],
    )
    def body(x_hbm, tab_hbm, gamma_hbm, beta_hbm, out_hbm,
             idx_all, r0, r1, r2, r3, g_v, b_v,
             g0, g1, g2, g3, o0, o1, o2, o3):
        wid = lax.axis_index("s") * NC + lax.axis_index("c")
        base = wid * rows_per_w
        bufs = (r0, r1, r2, r3)
        semg = (g0, g1, g2, g3)
        semo = (o0, o1, o2, o3)

        pltpu.sync_copy(gamma_hbm, g_v)
        pltpu.sync_copy(beta_hbm, b_v)
        pltpu.sync_copy(x_hbm.at[pl.ds(base, rows_per_w)], idx_all)
        gs = tuple(g_v[pl.ds(L16 * k, L16)] for k in range(NK))
        bs = tuple(b_v[pl.ds(L16 * k, L16)] for k in range(NK))

        def start_gather(c, buf, sem):
            pltpu.async_copy(
                tab_hbm.at[idx_all.at[pl.ds(c * CHUNK, CHUNK)]], buf, sem)

        start_gather(jnp.int32(0), bufs[0], semg[0])

        def super_body(ci, carry):
            for j in range(NBUF):
                c = ci * NBUF + j
                jn = (j + 1) % NBUF
                pltpu.make_async_copy(
                    tab_hbm.at[pl.ds(0, CHUNK)], bufs[j], semg[j]).wait()
                out_wait = lambda: pltpu.make_async_copy(
                    bufs[jn], out_hbm.at[pl.ds(0, CHUNK)], semo[jn]).wait()
                if j < NBUF - 1:
                    pl.when(ci > 0)(out_wait)
                else:
                    out_wait()
                cn = jnp.minimum(c + 1, nchunk - 1)
                start_gather(cn, bufs[jn], semg[jn])

                def ln_quad(p, c2, _buf=bufs[j]):
                    cgs, cbs = c2
                    for q in range(4):
                        _ln_rows(_buf, 4 * p + q, cgs, cbs)
                    return c2

                lax.fori_loop(0, CHUNK // 4, ln_quad, carry)
                pltpu.async_copy(
                    bufs[j], out_hbm.at[pl.ds(base + c * CHUNK, CHUNK)],
                    semo[j])
            return carry

        lax.fori_loop(0, nsuper, super_body, (gs, bs))

        for j in (1, 2, 3):
            pltpu.make_async_copy(
                bufs[j], out_hbm.at[pl.ds(0, CHUNK)], semo[j]).wait()
        pltpu.make_async_copy(
            tab_hbm.at[pl.ds(0, CHUNK)], bufs[0], semg[0]).wait()

    return body


def kernel(x, table, gamma, beta):
    bsz, seq = x.shape
    xi = x.T.reshape(-1).astype(jnp.int32)
    out = _make_sc_kernel(xi.shape[0])(xi, table, gamma, beta)
    return out.reshape(seq, bsz, D).transpose(1, 0, 2)
